# baseline (speedup 1.0000x reference)
import jax
import jax.numpy as jnp
from jax import lax
from jax.experimental import pallas as pl
from jax.experimental.pallas import tpu as pltpu
from jax.experimental.pallas import tpu_sc as plsc

B, S, D = 4096, 2, 64
NUM_ID, K = 100000, 4
NC, NS = 2, 16
NW = NC * NS
ROWS_PER_TILE = B // NW
SUB = 16
NQ = ROWS_PER_TILE // SUB
ROW = S * D
OUT_ROW = (S + K) * D
KS = S + K


def _rsqrt(x):
    i = plsc.bitcast(x, jnp.int32)
    y = plsc.bitcast(jnp.int32(0x5F3759DF) - (i >> 1), jnp.float32)
    for _ in range(2):
        y = y * (1.5 - 0.5 * x * y * y)
    return y


def _sc_body(feats_hbm, ids_hbm, f_out,
             table, ids_all, w_chunk, fs, fw, buf,
             fs_sems, fw_sems, out_sems, ids_sems):
    wid = lax.axis_index("s") * NC + lax.axis_index("c")
    base = wid * ROWS_PER_TILE
    lane = lax.iota(jnp.int32, 16)

    fs_pend = [None] * NQ
    fw_pend = [None] * NQ
    out_pend = [None] * NQ
    for q in range(2):
        fs_pend[q] = pltpu.async_copy(
            feats_hbm.at[pl.ds(base + q * SUB, SUB)], fs[q], fs_sems[q])
    IC = B // 4
    with jax.named_scope("p_ids"):
        ids_pend = [
            pltpu.async_copy(ids_hbm.at[pl.ds(h * IC, IC)],
                             ids_all.at[pl.ds(h * IC, IC)], ids_sems[h])
            for h in range(4)]

    rot = (lane + 1) & 15
    is15 = lane == 15

    def scat_body(k, c):
        j0 = k * 16
        idv = ids_all[pl.ds(j0, 16)]
        key = (idv << 12) | (lane + j0)
        skey = lax.sort(key)
        nxt = jnp.take_along_axis(skey, rot, axis=0,
                                  mode="promise_in_bounds")
        last = ((skey >> 12) != (nxt >> 12)) | is15
        plsc.store_scatter(table, [skey >> 12], skey & (B - 1), mask=last)
        return c
    with jax.named_scope("p_scat"):
        for h in range(4):
            ids_pend[h].wait()
            lax.fori_loop(h * (IC // 16), (h + 1) * (IC // 16),
                          scat_body, 0, unroll=4)

    def w_body(k, c):
        idx = ids_all[pl.ds(base + k * 16, 16)]
        w_chunk[pl.ds(k * 16, 16)] = plsc.load_gather(table, [idx])
        return c
    with jax.named_scope("p_w"):
        lax.fori_loop(0, ROWS_PER_TILE // 16, w_body, 0, unroll=4)

    def norm_group(vs):
        s = vs[0] * vs[0] + vs[1] * vs[1] + vs[2] * vs[2] + vs[3] * vs[3]
        tot = jnp.sum(s)
        x = jnp.maximum(lax.broadcast(tot, (16,)), 1e-24)
        return _rsqrt(x)

    for q in range(2):
        fw_pend[q] = pltpu.async_copy(
            feats_hbm.at[w_chunk.at[pl.ds(q * SUB, SUB)]], fw[q],
            fw_sems[q])

    for q in range(NQ):
        cur = q % 2
        rbase = base + q * SUB
        with jax.named_scope("p_wait_in"):
            fs_pend[q].wait()
            fw_pend[q].wait()
        fsq, fwq, bufq = fs[cur], fw[cur], buf[cur]
        if q >= 2:
            out_pend[q - 2].wait()

        def r_body(r, c2):
            for src, cols in ((fsq, (0,)), (fwq, (ROW, 2 * ROW))):
                for g in range(S):
                    vs = [src[r, pl.ds(g * D + k2 * 16, 16)]
                          for k2 in range(4)]
                    y = norm_group(vs)
                    for k2 in range(4):
                        o = vs[k2] * y
                        for c0 in cols:
                            bufq[r, pl.ds(c0 + g * D + k2 * 16, 16)] = o
            return c2
        with jax.named_scope("p_norm"):
            lax.fori_loop(0, SUB, r_body, 0, unroll=4)

        if q + 2 < NQ:
            fs_pend[q + 2] = pltpu.async_copy(
                feats_hbm.at[pl.ds(rbase + 2 * SUB, SUB)], fs[cur],
                fs_sems[cur])
            fw_pend[q + 2] = pltpu.async_copy(
                feats_hbm.at[w_chunk.at[pl.ds((q + 2) * SUB, SUB)]],
                fw[cur], fw_sems[cur])
        out_pend[q] = pltpu.async_copy(
            bufq, f_out.at[pl.ds(rbase, SUB)], out_sems[cur])

    with jax.named_scope("p_drain"):
        out_pend[NQ - 2].wait()
        out_pend[NQ - 1].wait()


@jax.jit
def _sc_call(feats2, ids):
    mesh = plsc.VectorSubcoreMesh(core_axis_name="c", subcore_axis_name="s")
    return pl.kernel(
        _sc_body,
        out_type=jax.ShapeDtypeStruct((B, OUT_ROW), jnp.float32),
        mesh=mesh,
        compiler_params=pltpu.CompilerParams(needs_layout_passes=False),
        scratch_types=[
            pltpu.VMEM((NUM_ID,), jnp.int32),
            pltpu.VMEM((B,), jnp.int32),
            pltpu.VMEM((ROWS_PER_TILE,), jnp.int32),
            [pltpu.VMEM((SUB, ROW), jnp.float32)] * 2,
            [pltpu.VMEM((SUB, ROW), jnp.float32)] * 2,
            [pltpu.VMEM((SUB, OUT_ROW), jnp.float32)] * 2,
            [pltpu.SemaphoreType.DMA] * 2,
            [pltpu.SemaphoreType.DMA] * 2,
            [pltpu.SemaphoreType.DMA] * 2,
            [pltpu.SemaphoreType.DMA] * 4,
        ],
    )(feats2, ids)


def kernel(feats, targets, mem):
    f_flat = _sc_call(feats.reshape(B, ROW), targets[:, 0])
    t = jnp.repeat(targets[:, :1], KS, axis=1)
    return f_flat.reshape(B, KS, D), t


# speedup vs baseline: 1.0118x; 1.0118x over previous
import jax
import jax.numpy as jnp
from jax import lax
from jax.experimental import pallas as pl
from jax.experimental.pallas import tpu as pltpu
from jax.experimental.pallas import tpu_sc as plsc

B, S, D = 4096, 2, 64
NUM_ID, K = 100000, 4
NC, NS = 2, 16
NW = NC * NS
ROWS_PER_TILE = B // NW
SUB = 16
NQ = ROWS_PER_TILE // SUB
ROW = S * D
OUT_ROW = (S + K) * D
KS = S + K


def _rsqrt(x):
    i = plsc.bitcast(x, jnp.int32)
    y = plsc.bitcast(jnp.int32(0x5F3759DF) - (i >> 1), jnp.float32)
    for _ in range(2):
        y = y * (1.5 - 0.5 * x * y * y)
    return y


def _sc_body(feats_hbm, ids_hbm, f_out,
             table, ids_all, w_chunk, fs, fw, buf,
             fs_sems, fw_sems, out_sems, ids_sems):
    wid = lax.axis_index("s") * NC + lax.axis_index("c")
    base = wid * ROWS_PER_TILE
    lane = lax.iota(jnp.int32, 16)

    fs_pend = [None] * NQ
    fw_pend = [None] * NQ
    out_pend = [None] * NQ
    for q in range(2):
        fs_pend[q] = pltpu.async_copy(
            feats_hbm.at[pl.ds(base + q * SUB, SUB)], fs[q], fs_sems[q])
    IC = B // 4
    with jax.named_scope("p_ids"):
        ids_pend = [
            pltpu.async_copy(ids_hbm.at[pl.ds(h * IC, IC)],
                             ids_all.at[pl.ds(h * IC, IC)], ids_sems[h])
            for h in range(4)]

    rot = (lane + 1) & 15
    is15 = lane == 15

    def sort_chunk(k):
        idv = ids_all[pl.ds(k * 16, 16)]
        return lax.sort((idv << 12) | (lane + k * 16))

    def emit_chunk(skey):
        nxt = jnp.take_along_axis(skey, rot, axis=0,
                                  mode="promise_in_bounds")
        last = ((skey >> 12) != (nxt >> 12)) | is15
        plsc.store_scatter(table, [skey >> 12], skey & (B - 1), mask=last)

    def scat_body(k, skey_prev):
        skey_k = sort_chunk(k)
        emit_chunk(skey_prev)
        return skey_k

    with jax.named_scope("p_scat"):
        ids_pend[0].wait()
        carry = sort_chunk(0)
        for h in range(4):
            if h + 1 < 4:
                ids_pend[h + 1].wait()
            lo, hi = h * (IC // 16), (h + 1) * (IC // 16)
            carry = lax.fori_loop(lo + 1 if h == 0 else lo, hi,
                                  scat_body, carry, unroll=4)
        emit_chunk(carry)

    def w_body(k, c):
        idx = ids_all[pl.ds(base + k * 16, 16)]
        w_chunk[pl.ds(k * 16, 16)] = plsc.load_gather(table, [idx])
        return c
    with jax.named_scope("p_w"):
        lax.fori_loop(0, ROWS_PER_TILE // 16, w_body, 0, unroll=4)

    bfly = [lane ^ (1 << b) for b in range(4)]

    def norm_group(vs):
        s = vs[0] * vs[0] + vs[1] * vs[1] + vs[2] * vs[2] + vs[3] * vs[3]
        for perm in bfly:
            s = s + jnp.take_along_axis(s, perm, axis=0,
                                        mode="promise_in_bounds")
        x = jnp.maximum(s, 1e-24)
        return _rsqrt(x)

    for q in range(2):
        fw_pend[q] = pltpu.async_copy(
            feats_hbm.at[w_chunk.at[pl.ds(q * SUB, SUB)]], fw[q],
            fw_sems[q])

    for q in range(NQ):
        cur = q % 2
        rbase = base + q * SUB
        with jax.named_scope("p_wait_in"):
            fs_pend[q].wait()
            fw_pend[q].wait()
        fsq, fwq, bufq = fs[cur], fw[cur], buf[cur]
        if q >= 2:
            out_pend[q - 2].wait()

        def r_body(r, c2):
            for src, cols in ((fsq, (0,)), (fwq, (ROW, 2 * ROW))):
                for g in range(S):
                    vs = [src[r, pl.ds(g * D + k2 * 16, 16)]
                          for k2 in range(4)]
                    y = norm_group(vs)
                    for k2 in range(4):
                        o = vs[k2] * y
                        for c0 in cols:
                            bufq[r, pl.ds(c0 + g * D + k2 * 16, 16)] = o
            return c2
        with jax.named_scope("p_norm"):
            lax.fori_loop(0, SUB, r_body, 0, unroll=4)

        if q + 2 < NQ:
            fs_pend[q + 2] = pltpu.async_copy(
                feats_hbm.at[pl.ds(rbase + 2 * SUB, SUB)], fs[cur],
                fs_sems[cur])
            fw_pend[q + 2] = pltpu.async_copy(
                feats_hbm.at[w_chunk.at[pl.ds((q + 2) * SUB, SUB)]],
                fw[cur], fw_sems[cur])
        out_pend[q] = pltpu.async_copy(
            bufq, f_out.at[pl.ds(rbase, SUB)], out_sems[cur])

    with jax.named_scope("p_drain"):
        out_pend[NQ - 2].wait()
        out_pend[NQ - 1].wait()


@jax.jit
def _sc_call(feats2, ids):
    mesh = plsc.VectorSubcoreMesh(core_axis_name="c", subcore_axis_name="s")
    return pl.kernel(
        _sc_body,
        out_type=jax.ShapeDtypeStruct((B, OUT_ROW), jnp.float32),
        mesh=mesh,
        compiler_params=pltpu.CompilerParams(needs_layout_passes=False),
        scratch_types=[
            pltpu.VMEM((NUM_ID,), jnp.int32),
            pltpu.VMEM((B,), jnp.int32),
            pltpu.VMEM((ROWS_PER_TILE,), jnp.int32),
            [pltpu.VMEM((SUB, ROW), jnp.float32)] * 2,
            [pltpu.VMEM((SUB, ROW), jnp.float32)] * 2,
            [pltpu.VMEM((SUB, OUT_ROW), jnp.float32)] * 2,
            [pltpu.SemaphoreType.DMA] * 2,
            [pltpu.SemaphoreType.DMA] * 2,
            [pltpu.SemaphoreType.DMA] * 2,
            [pltpu.SemaphoreType.DMA] * 4,
        ],
    )(feats2, ids)


def kernel(feats, targets, mem):
    f_flat = _sc_call(feats.reshape(B, ROW), targets[:, 0])
    t = jnp.repeat(targets[:, :1], KS, axis=1)
    return f_flat.reshape(B, KS, D), t


# speedup vs baseline: 1.3793x; 1.3632x over previous
import jax
import jax.numpy as jnp
from jax import lax
from jax.experimental import pallas as pl
from jax.experimental.pallas import tpu as pltpu
from jax.experimental.pallas import tpu_sc as plsc

B, S, D = 4096, 2, 64
NUM_ID, K = 100000, 4
NC, NS = 2, 16
NW = NC * NS
ROWS_PER_TILE = B // NW
SUB = 16
NQ = ROWS_PER_TILE // SUB
ROW = S * D
OUT_ROW = (S + K) * D
KS = S + K


def _rsqrt(x):
    i = plsc.bitcast(x, jnp.int32)
    y = plsc.bitcast(jnp.int32(0x5F3759DF) - (i >> 1), jnp.float32)
    for _ in range(2):
        y = y * (1.5 - 0.5 * x * y * y)
    return y


def _sc_body(feats_hbm, ids_hbm, f_out,
             table, ids_all, w_chunk, fs, fw, buf,
             fs_sems, fw_sems, out_sems, ids_sems):
    wid = lax.axis_index("s") * NC + lax.axis_index("c")
    base = wid * ROWS_PER_TILE
    lane = lax.iota(jnp.int32, 16)

    fs_pend = [None] * NQ
    fw_pend = [None] * NQ
    out_pend = [None] * NQ
    for q in range(2):
        fs_pend[q] = pltpu.async_copy(
            feats_hbm.at[pl.ds(base + q * SUB, SUB)], fs[q], fs_sems[q])
    IC = B // 4
    with jax.named_scope("p_ids"):
        ids_pend = [
            pltpu.async_copy(ids_hbm.at[pl.ds(h * IC, IC)],
                             ids_all.at[pl.ds(h * IC, IC)], ids_sems[h])
            for h in range(4)]

    rot = (lane + 1) & 15
    is15 = lane == 15

    def sort_chunk(k):
        idv = ids_all[pl.ds(k * 16, 16)]
        return lax.sort((idv << 12) | (lane + k * 16))

    def emit_chunk(skey):
        nxt = jnp.take_along_axis(skey, rot, axis=0,
                                  mode="promise_in_bounds")
        last = ((skey >> 12) != (nxt >> 12)) | is15
        plsc.store_scatter(table, [skey >> 12], skey & (B - 1), mask=last)

    def scat_body(k, skey_prev):
        skey_k = sort_chunk(k)
        emit_chunk(skey_prev)
        return skey_k

    with jax.named_scope("p_scat"):
        ids_pend[0].wait()
        carry = sort_chunk(0)
        for h in range(4):
            if h + 1 < 4:
                ids_pend[h + 1].wait()
            lo, hi = h * (IC // 16), (h + 1) * (IC // 16)
            carry = lax.fori_loop(lo + 1 if h == 0 else lo, hi,
                                  scat_body, carry, unroll=4)
        emit_chunk(carry)

    def w_body(k, c):
        idx = ids_all[pl.ds(base + k * 16, 16)]
        w_chunk[pl.ds(k * 16, 16)] = plsc.load_gather(table, [idx])
        return c
    with jax.named_scope("p_w"):
        lax.fori_loop(0, ROWS_PER_TILE // 16, w_body, 0, unroll=4)

    bfly = [lane ^ (1 << b) for b in range(4)]

    for q in range(2):
        fw_pend[q] = pltpu.async_copy(
            feats_hbm.at[w_chunk.at[pl.ds(q * SUB, SUB)]], fw[q],
            fw_sems[q])

    for q in range(NQ):
        cur = q % 2
        rbase = base + q * SUB
        with jax.named_scope("p_wait_in"):
            fs_pend[q].wait()
            fw_pend[q].wait()
        fsq, fwq, bufq = fs[cur], fw[cur], buf[cur]
        if q >= 2:
            out_pend[q - 2].wait()

        grp = [(fsq, 0, (0,)), (fsq, 1, (0,)),
               (fwq, 0, (ROW, 2 * ROW)), (fwq, 1, (ROW, 2 * ROW))]
        NG = len(grp)
        rng = range(NG)

        def r_body(r, c2):
            va = [[sr[r, pl.ds(g * D + k2 * 16, 16)] for k2 in range(4)]
                  for (sr, g, _) in grp]
            s = [va[i][0] * va[i][0] for i in rng]
            for k2 in (1, 2, 3):
                m = [va[i][k2] * va[i][k2] for i in rng]
                s = [s[i] + m[i] for i in rng]
            for perm in bfly:
                t = [jnp.take_along_axis(s[i], perm, axis=0,
                                         mode="promise_in_bounds")
                     for i in rng]
                s = [s[i] + t[i] for i in rng]
            x = [jnp.maximum(s[i], 1e-24) for i in rng]
            iv = [plsc.bitcast(x[i], jnp.int32) for i in rng]
            y = [plsc.bitcast(jnp.int32(0x5F3759DF) - (iv[i] >> 1),
                              jnp.float32) for i in rng]
            h = [0.5 * x[i] for i in rng]
            for _ in range(2):
                t = [y[i] * y[i] for i in rng]
                t = [h[i] * t[i] for i in rng]
                t = [1.5 - t[i] for i in rng]
                y = [y[i] * t[i] for i in rng]
            for i, (sr, g, cols) in enumerate(grp):
                for k2 in range(4):
                    o = va[i][k2] * y[i]
                    for c0 in cols:
                        bufq[r, pl.ds(c0 + g * D + k2 * 16, 16)] = o
            return c2
        with jax.named_scope("p_norm"):
            lax.fori_loop(0, SUB, r_body, 0)

        if q + 2 < NQ:
            fs_pend[q + 2] = pltpu.async_copy(
                feats_hbm.at[pl.ds(rbase + 2 * SUB, SUB)], fs[cur],
                fs_sems[cur])
            fw_pend[q + 2] = pltpu.async_copy(
                feats_hbm.at[w_chunk.at[pl.ds((q + 2) * SUB, SUB)]],
                fw[cur], fw_sems[cur])
        out_pend[q] = pltpu.async_copy(
            bufq, f_out.at[pl.ds(rbase, SUB)], out_sems[cur])

    with jax.named_scope("p_drain"):
        out_pend[NQ - 2].wait()
        out_pend[NQ - 1].wait()


@jax.jit
def _sc_call(feats2, ids):
    mesh = plsc.VectorSubcoreMesh(core_axis_name="c", subcore_axis_name="s")
    return pl.kernel(
        _sc_body,
        out_type=jax.ShapeDtypeStruct((B, OUT_ROW), jnp.float32),
        mesh=mesh,
        compiler_params=pltpu.CompilerParams(needs_layout_passes=False),
        scratch_types=[
            pltpu.VMEM((NUM_ID,), jnp.int32),
            pltpu.VMEM((B,), jnp.int32),
            pltpu.VMEM((ROWS_PER_TILE,), jnp.int32),
            [pltpu.VMEM((SUB, ROW), jnp.float32)] * 2,
            [pltpu.VMEM((SUB, ROW), jnp.float32)] * 2,
            [pltpu.VMEM((SUB, OUT_ROW), jnp.float32)] * 2,
            [pltpu.SemaphoreType.DMA] * 2,
            [pltpu.SemaphoreType.DMA] * 2,
            [pltpu.SemaphoreType.DMA] * 2,
            [pltpu.SemaphoreType.DMA] * 4,
        ],
    )(feats2, ids)


def kernel(feats, targets, mem):
    f_flat = _sc_call(feats.reshape(B, ROW), targets[:, 0])
    t = jnp.repeat(targets[:, :1], KS, axis=1)
    return f_flat.reshape(B, KS, D), t


# speedup vs baseline: 1.3848x; 1.0040x over previous
import jax
import jax.numpy as jnp
from jax import lax
from jax.experimental import pallas as pl
from jax.experimental.pallas import tpu as pltpu
from jax.experimental.pallas import tpu_sc as plsc

B, S, D = 4096, 2, 64
NUM_ID, K = 100000, 4
NC, NS = 2, 16
NW = NC * NS
ROWS_PER_TILE = B // NW
SUB = 16
NQ = ROWS_PER_TILE // SUB
ROW = S * D
OUT_ROW = (S + K) * D
KS = S + K


def _rsqrt(x):
    i = plsc.bitcast(x, jnp.int32)
    y = plsc.bitcast(jnp.int32(0x5F3759DF) - (i >> 1), jnp.float32)
    for _ in range(2):
        y = y * (1.5 - 0.5 * x * y * y)
    return y


def _sc_body(feats_hbm, ids_hbm, f_out,
             table, ids_all, w_chunk, fs, fw, buf,
             fs_sems, fw_sems, out_sems, ids_sems):
    wid = lax.axis_index("s") * NC + lax.axis_index("c")
    base = wid * ROWS_PER_TILE
    lane = lax.iota(jnp.int32, 16)

    fs_pend = [None] * NQ
    fw_pend = [None] * NQ
    out_pend = [None] * NQ
    for q in range(2):
        fs_pend[q] = pltpu.async_copy(
            feats_hbm.at[pl.ds(base + q * SUB, SUB)], fs[q], fs_sems[q])
    IC = B // 4
    with jax.named_scope("p_ids"):
        ids_pend = [
            pltpu.async_copy(ids_hbm.at[pl.ds(h * IC, IC)],
                             ids_all.at[pl.ds(h * IC, IC)], ids_sems[h])
            for h in range(4)]

    rot = (lane + 1) & 15
    is15 = lane == 15

    def sort_chunk(k):
        idv = ids_all[pl.ds(k * 16, 16)]
        return lax.sort((idv << 12) | (lane + k * 16))

    def emit_chunk(skey):
        nxt = jnp.take_along_axis(skey, rot, axis=0,
                                  mode="promise_in_bounds")
        last = ((skey >> 12) != (nxt >> 12)) | is15
        plsc.store_scatter(table, [skey >> 12], skey & (B - 1), mask=last)

    def scat_body(k, skey_prev):
        skey_k = sort_chunk(k)
        emit_chunk(skey_prev)
        return skey_k

    with jax.named_scope("p_scat"):
        ids_pend[0].wait()
        carry = sort_chunk(0)
        for h in range(4):
            if h + 1 < 4:
                ids_pend[h + 1].wait()
            lo, hi = h * (IC // 16), (h + 1) * (IC // 16)
            carry = lax.fori_loop(lo + 1 if h == 0 else lo, hi,
                                  scat_body, carry, unroll=2)
        emit_chunk(carry)

    def w_body(k, c):
        idx = ids_all[pl.ds(base + k * 16, 16)]
        w_chunk[pl.ds(k * 16, 16)] = plsc.load_gather(table, [idx])
        return c
    with jax.named_scope("p_w"):
        lax.fori_loop(0, ROWS_PER_TILE // 16, w_body, 0, unroll=4)

    bfly = [lane ^ (1 << b) for b in range(4)]

    for q in range(2):
        fw_pend[q] = pltpu.async_copy(
            feats_hbm.at[w_chunk.at[pl.ds(q * SUB, SUB)]], fw[q],
            fw_sems[q])

    for q in range(NQ):
        cur = q % 2
        rbase = base + q * SUB
        with jax.named_scope("p_wait_in"):
            fs_pend[q].wait()
            fw_pend[q].wait()
        fsq, fwq, bufq = fs[cur], fw[cur], buf[cur]
        if q >= 2:
            out_pend[q - 2].wait()

        grp = [(fsq, 0, (0,)), (fsq, 1, (0,)),
               (fwq, 0, (ROW, 2 * ROW)), (fwq, 1, (ROW, 2 * ROW))]
        NG = len(grp)
        rng = range(NG)

        def r_body(r, c2):
            va = [[sr[r, pl.ds(g * D + k2 * 16, 16)] for k2 in range(4)]
                  for (sr, g, _) in grp]
            s = [va[i][0] * va[i][0] for i in rng]
            for k2 in (1, 2, 3):
                m = [va[i][k2] * va[i][k2] for i in rng]
                s = [s[i] + m[i] for i in rng]
            for perm in bfly:
                t = [jnp.take_along_axis(s[i], perm, axis=0,
                                         mode="promise_in_bounds")
                     for i in rng]
                s = [s[i] + t[i] for i in rng]
            x = [jnp.maximum(s[i], 1e-24) for i in rng]
            iv = [plsc.bitcast(x[i], jnp.int32) for i in rng]
            y = [plsc.bitcast(jnp.int32(0x5F3759DF) - (iv[i] >> 1),
                              jnp.float32) for i in rng]
            h = [0.5 * x[i] for i in rng]
            for _ in range(2):
                t = [y[i] * y[i] for i in rng]
                t = [h[i] * t[i] for i in rng]
                t = [1.5 - t[i] for i in rng]
                y = [y[i] * t[i] for i in rng]
            for i, (sr, g, cols) in enumerate(grp):
                for k2 in range(4):
                    o = va[i][k2] * y[i]
                    for c0 in cols:
                        bufq[r, pl.ds(c0 + g * D + k2 * 16, 16)] = o
            return c2
        with jax.named_scope("p_norm"):
            lax.fori_loop(0, SUB, r_body, 0)

        if q + 2 < NQ:
            fs_pend[q + 2] = pltpu.async_copy(
                feats_hbm.at[pl.ds(rbase + 2 * SUB, SUB)], fs[cur],
                fs_sems[cur])
            fw_pend[q + 2] = pltpu.async_copy(
                feats_hbm.at[w_chunk.at[pl.ds((q + 2) * SUB, SUB)]],
                fw[cur], fw_sems[cur])
        out_pend[q] = pltpu.async_copy(
            bufq, f_out.at[pl.ds(rbase, SUB)], out_sems[cur])

    with jax.named_scope("p_drain"):
        out_pend[NQ - 2].wait()
        out_pend[NQ - 1].wait()


@jax.jit
def _sc_call(feats2, ids):
    mesh = plsc.VectorSubcoreMesh(core_axis_name="c", subcore_axis_name="s")
    return pl.kernel(
        _sc_body,
        out_type=jax.ShapeDtypeStruct((B, OUT_ROW), jnp.float32),
        mesh=mesh,
        compiler_params=pltpu.CompilerParams(needs_layout_passes=False),
        scratch_types=[
            pltpu.VMEM((NUM_ID,), jnp.int32),
            pltpu.VMEM((B,), jnp.int32),
            pltpu.VMEM((ROWS_PER_TILE,), jnp.int32),
            [pltpu.VMEM((SUB, ROW), jnp.float32)] * 2,
            [pltpu.VMEM((SUB, ROW), jnp.float32)] * 2,
            [pltpu.VMEM((SUB, OUT_ROW), jnp.float32)] * 2,
            [pltpu.SemaphoreType.DMA] * 2,
            [pltpu.SemaphoreType.DMA] * 2,
            [pltpu.SemaphoreType.DMA] * 2,
            [pltpu.SemaphoreType.DMA] * 4,
        ],
    )(feats2, ids)


def kernel(feats, targets, mem):
    f_flat = _sc_call(feats.reshape(B, ROW), targets[:, 0])
    t = jnp.repeat(targets[:, :1], KS, axis=1)
    return f_flat.reshape(B, KS, D), t
